# baseline (speedup 1.0000x reference)
import functools

import jax
import jax.numpy as jnp
from jax import lax
from jax.experimental import pallas as pl
from jax.experimental.pallas import tpu as pltpu
from jax.experimental.pallas import tpu_sc as plsc

B, F, D = 4096, 100, 128
NC, NS = 2, 16
NW = NC * NS
RPW = B // NW
NSLOT = 2
G = F // (2 * NSLOT)

_mesh = plsc.VectorSubcoreMesh(core_axis_name="c", subcore_axis_name="s")


@functools.partial(
    pl.kernel,
    mesh=_mesh,
    out_type=jax.ShapeDtypeStruct((F, B, D), jnp.float32),
    scratch_types=(
        [pltpu.VMEM_SHARED((NS, NSLOT, RPW, D), jnp.float32)]
        + [pltpu.VMEM((RPW, D), jnp.float32) for _ in range(NSLOT)]
        + [pltpu.SemaphoreType.DMA for _ in range(4 * NSLOT)]
    ),
)
def _rev_copy(x_hbm, out_hbm, shared, *refs):
    vbuf = refs[0:NSLOT]
    rsem = refs[NSLOT:3 * NSLOT]
    wsem = refs[3 * NSLOT:5 * NSLOT]
    sid = lax.axis_index("s")
    wid = sid * NC + lax.axis_index("c")
    r0 = wid * RPW

    def _buf(k):
        return shared.at[sid, k // 2] if k % 2 == 0 else vbuf[k // 2]

    def start_read(k, f):
        pltpu.async_copy(x_hbm.at[F - 1 - f].at[pl.ds(r0, RPW)],
                         _buf(k), rsem[k])

    def wait_read(k):
        pltpu.make_async_copy(x_hbm.at[0].at[pl.ds(r0, RPW)],
                              _buf(k), rsem[k]).wait()

    def start_write(k, f):
        pltpu.async_copy(_buf(k), out_hbm.at[f].at[pl.ds(r0, RPW)],
                         wsem[k])

    def wait_write(k):
        pltpu.make_async_copy(_buf(k), out_hbm.at[0].at[pl.ds(r0, RPW)],
                              wsem[k]).wait()

    A = list(range(NSLOT))
    Bset = list(range(NSLOT, 2 * NSLOT))

    for i, k in enumerate(A):
        start_read(k, i)

    def body(h, carry):
        f0 = h * 2 * NSLOT
        for k in A:
            wait_read(k)

        @pl.when(h > 0)
        def _wb():
            for k in Bset:
                wait_write(k)

        for i, k in enumerate(A):
            start_write(k, f0 + i)
        for i, k in enumerate(Bset):
            start_read(k, f0 + NSLOT + i)
        for k in Bset:
            wait_read(k)
        for k in A:
            wait_write(k)
        for i, k in enumerate(Bset):
            start_write(k, f0 + NSLOT + i)

        @pl.when(h < G - 1)
        def _ra():
            for i, k in enumerate(A):
                start_read(k, f0 + 2 * NSLOT + i)

        return carry

    lax.fori_loop(0, G, body, 0)
    for k in Bset:
        wait_write(k)


def kernel(x):
    out_t = _rev_copy(x.transpose(1, 0, 2))
    return out_t.transpose(1, 0, 2)


# speedup vs baseline: 1.0185x; 1.0185x over previous
import functools

import jax
import jax.numpy as jnp
from jax import lax
from jax.experimental import pallas as pl
from jax.experimental.pallas import tpu as pltpu
from jax.experimental.pallas import tpu_sc as plsc

B, F, D = 4096, 100, 128
NC, NS = 2, 16
NW = NC * NS
RPW = B // NW
NSLOT = 2
G = F // (2 * NSLOT)

_mesh = plsc.VectorSubcoreMesh(core_axis_name="c", subcore_axis_name="s")


@functools.partial(
    pl.kernel,
    mesh=_mesh,
    out_type=jax.ShapeDtypeStruct((F, B, D), jnp.float32),
    scratch_types=(
        [pltpu.VMEM_SHARED((NS, 2 * NSLOT, RPW, D), jnp.float32)]
        + [pltpu.SemaphoreType.DMA for _ in range(4 * NSLOT)]
    ),
)
def _rev_copy(x_hbm, out_hbm, shared, *refs):
    rsem = refs[0:2 * NSLOT]
    wsem = refs[2 * NSLOT:4 * NSLOT]
    sid = lax.axis_index("s")
    wid = sid * NC + lax.axis_index("c")
    r0 = wid * RPW

    def _buf(k):
        return shared.at[sid, k]

    def start_read(k, f):
        pltpu.async_copy(x_hbm.at[F - 1 - f].at[pl.ds(r0, RPW)],
                         _buf(k), rsem[k])

    def wait_read(k):
        pltpu.make_async_copy(x_hbm.at[0].at[pl.ds(r0, RPW)],
                              _buf(k), rsem[k]).wait()

    def start_write(k, f):
        pltpu.async_copy(_buf(k), out_hbm.at[f].at[pl.ds(r0, RPW)],
                         wsem[k])

    def wait_write(k):
        pltpu.make_async_copy(_buf(k), out_hbm.at[0].at[pl.ds(r0, RPW)],
                              wsem[k]).wait()

    A = list(range(NSLOT))
    Bset = list(range(NSLOT, 2 * NSLOT))

    for i, k in enumerate(A):
        start_read(k, i)

    def body(h, carry):
        f0 = h * 2 * NSLOT
        for k in A:
            wait_read(k)

        @pl.when(h > 0)
        def _wb():
            for k in Bset:
                wait_write(k)

        for i, k in enumerate(A):
            start_write(k, f0 + i)
        for i, k in enumerate(Bset):
            start_read(k, f0 + NSLOT + i)
        for k in Bset:
            wait_read(k)
        for k in A:
            wait_write(k)
        for i, k in enumerate(Bset):
            start_write(k, f0 + NSLOT + i)

        @pl.when(h < G - 1)
        def _ra():
            for i, k in enumerate(A):
                start_read(k, f0 + 2 * NSLOT + i)

        return carry

    lax.fori_loop(0, G, body, 0)
    for k in Bset:
        wait_write(k)


def kernel(x):
    out_t = _rev_copy(x.transpose(1, 0, 2))
    return out_t.transpose(1, 0, 2)


# speedup vs baseline: 1.0243x; 1.0057x over previous
import functools

import jax
import jax.numpy as jnp
from jax import lax
from jax.experimental import pallas as pl
from jax.experimental.pallas import tpu as pltpu
from jax.experimental.pallas import tpu_sc as plsc

B, F, D = 4096, 100, 128
NC, NS = 2, 16
NW = NC * NS
NRG = 16
RPW = B // NRG
FPW = F // 2
G = FPW // 2

_mesh = plsc.VectorSubcoreMesh(core_axis_name="c", subcore_axis_name="s")


@functools.partial(
    pl.kernel,
    mesh=_mesh,
    out_type=jax.ShapeDtypeStruct((F, B, D), jnp.float32),
    scratch_types=(
        [pltpu.VMEM_SHARED((NS, 2, RPW, D), jnp.float32)]
        + [pltpu.SemaphoreType.DMA for _ in range(4)]
    ),
)
def _rev_copy(x_hbm, out_hbm, shared, *sems):
    rsem = sems[0:2]
    wsem = sems[2:4]
    sid = lax.axis_index("s")
    wid = sid * NC + lax.axis_index("c")
    r0 = (wid % NRG) * RPW
    fpar = wid // NRG

    def start_read(k, j):
        f = fpar + 2 * j
        pltpu.async_copy(x_hbm.at[F - 1 - f].at[pl.ds(r0, RPW)],
                         shared.at[sid, k], rsem[k])

    def wait_read(k):
        pltpu.make_async_copy(x_hbm.at[0].at[pl.ds(r0, RPW)],
                              shared.at[sid, k], rsem[k]).wait()

    def start_write(k, j):
        f = fpar + 2 * j
        pltpu.async_copy(shared.at[sid, k], out_hbm.at[f].at[pl.ds(r0, RPW)],
                         wsem[k])

    def wait_write(k):
        pltpu.make_async_copy(shared.at[sid, k],
                              out_hbm.at[0].at[pl.ds(r0, RPW)],
                              wsem[k]).wait()

    start_read(0, 0)

    def body(h, carry):
        j0 = 2 * h
        wait_read(0)

        @pl.when(h > 0)
        def _wb():
            wait_write(1)

        start_write(0, j0)
        start_read(1, j0 + 1)
        wait_read(1)
        wait_write(0)
        start_write(1, j0 + 1)

        @pl.when(h < G - 1)
        def _ra():
            start_read(0, j0 + 2)

        return carry

    lax.fori_loop(0, G, body, 0)
    wait_write(1)


def kernel(x):
    out_t = _rev_copy(x.transpose(1, 0, 2))
    return out_t.transpose(1, 0, 2)


# speedup vs baseline: 1.0257x; 1.0013x over previous
import functools

import jax
import jax.numpy as jnp
from jax import lax
from jax.experimental import pallas as pl
from jax.experimental.pallas import tpu as pltpu
from jax.experimental.pallas import tpu_sc as plsc

B, F, D = 4096, 100, 128
NC, NS = 2, 16
NW = NC * NS
NRG = 16
RPW = B // NRG
FPW = F // 2
G = 16

_mesh = plsc.VectorSubcoreMesh(core_axis_name="c", subcore_axis_name="s")


@functools.partial(
    pl.kernel,
    mesh=_mesh,
    out_type=jax.ShapeDtypeStruct((F, B, D), jnp.float32),
    scratch_types=(
        [pltpu.VMEM_SHARED((NS, 3, RPW, D), jnp.float32)]
        + [pltpu.SemaphoreType.DMA for _ in range(6)]
    ),
)
def _rev_copy(x_hbm, out_hbm, shared, *sems):
    rsem = sems[0:3]
    wsem = sems[3:6]
    sid = lax.axis_index("s")
    wid = sid * NC + lax.axis_index("c")
    r0 = (wid % NRG) * RPW
    fpar = wid // NRG

    def start_read(k, j):
        f = fpar + 2 * j
        pltpu.async_copy(x_hbm.at[F - 1 - f].at[pl.ds(r0, RPW)],
                         shared.at[sid, k], rsem[k])

    def wait_read(k):
        pltpu.make_async_copy(x_hbm.at[0].at[pl.ds(r0, RPW)],
                              shared.at[sid, k], rsem[k]).wait()

    def start_write(k, j):
        f = fpar + 2 * j
        pltpu.async_copy(shared.at[sid, k], out_hbm.at[f].at[pl.ds(r0, RPW)],
                         wsem[k])

    def wait_write(k):
        pltpu.make_async_copy(shared.at[sid, k],
                              out_hbm.at[0].at[pl.ds(r0, RPW)],
                              wsem[k]).wait()

    start_read(0, 0)
    start_read(1, 1)

    def body(g, carry):
        j0 = 3 * g
        wait_read(0)

        @pl.when(g > 0)
        def _wb():
            wait_write(2)

        start_write(0, j0)
        start_read(2, j0 + 2)

        wait_read(1)
        wait_write(0)
        start_write(1, j0 + 1)
        start_read(0, j0 + 3)

        wait_read(2)
        wait_write(1)
        start_write(2, j0 + 2)
        start_read(1, j0 + 4)

        return carry

    lax.fori_loop(0, G, body, 0)
    wait_read(0)
    wait_write(2)
    start_write(0, FPW - 2)
    wait_read(1)
    wait_write(0)
    start_write(1, FPW - 1)
    wait_write(1)


def kernel(x):
    out_t = _rev_copy(x.transpose(1, 0, 2))
    return out_t.transpose(1, 0, 2)
